# baseline (speedup 1.0000x reference)
import functools

import jax
import jax.numpy as jnp
from jax import lax
from jax.experimental import pallas as pl
from jax.experimental.pallas import tpu as pltpu
from jax.experimental.pallas import tpu_sc as plsc

_LEAD = 128
_R = 2048
_C = 128
_BLK = 8
_DST = (34, 0, 1)
_SRC = (50, 1, 0)

_sc_mesh = plsc.VectorSubcoreMesh(core_axis_name="c", subcore_axis_name="s")


@functools.partial(
    pl.kernel,
    mesh=_sc_mesh,
    out_type=jax.ShapeDtypeStruct((8, _C), jnp.float32),
    scratch_types=[pltpu.VMEM((16,), jnp.float32)],
)
def _sc_get(x_hbm, patch_hbm, v):
    @pl.when((lax.axis_index("c") == 0) & (lax.axis_index("s") == 0))
    def _():
        pltpu.sync_copy(x_hbm.at[_SRC[0], _SRC[1], pl.ds(0, 16)], v)
        pltpu.sync_copy(v, patch_hbm.at[0, pl.ds(0, 16)])


def _ew_kernel(x_ref, patch_ref, o_ref):
    i = pl.program_id(0)
    dst_blk = _DST[0] // _BLK

    @pl.when(i == dst_blk)
    def _():
        s = 2.0 * jnp.log(patch_ref[0, _SRC[2]])
        d0 = jax.lax.broadcasted_iota(jnp.int32, (_BLK, _R, _C), 0)
        d1 = jax.lax.broadcasted_iota(jnp.int32, (_BLK, _R, _C), 1)
        d2 = jax.lax.broadcasted_iota(jnp.int32, (_BLK, _R, _C), 2)
        mask = (
            (d0 == _DST[0] - dst_blk * _BLK) & (d1 == _DST[1]) & (d2 == _DST[2])
        )
        o_ref[...] = jnp.where(mask, s, 2.0 * jnp.log(x_ref[...]))

    @pl.when(i != dst_blk)
    def _():
        o_ref[...] = 2.0 * jnp.log(x_ref[...])


def kernel(x):
    xr = x.reshape(_LEAD, _R, _C)
    patch = _sc_get(xr)
    out = pl.pallas_call(
        _ew_kernel,
        grid=(_LEAD // _BLK,),
        in_specs=[
            pl.BlockSpec((_BLK, _R, _C), lambda i: (i, 0, 0)),
            pl.BlockSpec((8, _C), lambda i: (0, 0)),
        ],
        out_specs=pl.BlockSpec((_BLK, _R, _C), lambda i: (i, 0, 0)),
        out_shape=jax.ShapeDtypeStruct((_LEAD, _R, _C), x.dtype),
    )(xr, patch)
    return out.reshape(x.shape)


# speedup vs baseline: 1.0226x; 1.0226x over previous
import functools

import jax
import jax.numpy as jnp
from jax import lax
from jax.experimental import pallas as pl
from jax.experimental.pallas import tpu as pltpu
from jax.experimental.pallas import tpu_sc as plsc

_LEAD = 128
_R = 2048
_C = 128
_BLK = 8
_DST = (34, 0, 1)
_SRC = (50, 1, 0)

_sc_mesh = plsc.VectorSubcoreMesh(core_axis_name="c", subcore_axis_name="s")


@functools.partial(
    pl.kernel,
    mesh=_sc_mesh,
    out_type=jax.ShapeDtypeStruct((8, _C), jnp.float32),
    scratch_types=[pltpu.VMEM((16,), jnp.float32)],
)
def _sc_get(x_hbm, patch_hbm, v):
    @pl.when((lax.axis_index("c") == 0) & (lax.axis_index("s") == 0))
    def _():
        pltpu.sync_copy(x_hbm.at[_SRC[0], _SRC[1], pl.ds(0, 16)], v)
        pltpu.sync_copy(v, patch_hbm.at[0, pl.ds(0, 16)])


def _ew_kernel(x_ref, o_ref):
    o_ref[...] = 2.0 * jnp.log(x_ref[...])


def _apply_kernel(y_ref, patch_ref, o_ref):
    s = 2.0 * jnp.log(patch_ref[0, _SRC[2]])
    d1 = jax.lax.broadcasted_iota(jnp.int32, (1, 8, _C), 1)
    d2 = jax.lax.broadcasted_iota(jnp.int32, (1, 8, _C), 2)
    mask = (d1 == _DST[1]) & (d2 == _DST[2])
    o_ref[...] = jnp.where(mask, s, y_ref[...])


def kernel(x):
    xr = x.reshape(_LEAD, _R, _C)
    patch = _sc_get(xr)
    y = pl.pallas_call(
        _ew_kernel,
        grid=(_LEAD // _BLK,),
        in_specs=[pl.BlockSpec((_BLK, _R, _C), lambda i: (i, 0, 0))],
        out_specs=pl.BlockSpec((_BLK, _R, _C), lambda i: (i, 0, 0)),
        out_shape=jax.ShapeDtypeStruct((_LEAD, _R, _C), x.dtype),
    )(xr)
    out = pl.pallas_call(
        _apply_kernel,
        grid=(1,),
        in_specs=[
            pl.BlockSpec((1, 8, _C), lambda i: (_DST[0], 0, 0)),
            pl.BlockSpec((8, _C), lambda i: (0, 0)),
        ],
        out_specs=pl.BlockSpec((1, 8, _C), lambda i: (_DST[0], 0, 0)),
        out_shape=jax.ShapeDtypeStruct((_LEAD, _R, _C), x.dtype),
        input_output_aliases={0: 0},
    )(y, patch)
    return out.reshape(x.shape)


# speedup vs baseline: 1.2314x; 1.2041x over previous
import jax
import jax.numpy as jnp
from jax.experimental import pallas as pl

_LEAD = 128
_R = 2048
_C = 128
_BLK = 8
_DST = (34, 0, 1)
_SRC = (50, 1, 0)


def _ew_kernel(x_ref, src_ref, o_ref):
    i = pl.program_id(0)
    dst_blk = _DST[0] // _BLK

    @pl.when(i == dst_blk)
    def _():
        s = 2.0 * jnp.log(src_ref[0, _SRC[1], _SRC[2]])
        d0 = jax.lax.broadcasted_iota(jnp.int32, (_BLK, _R, _C), 0)
        d1 = jax.lax.broadcasted_iota(jnp.int32, (_BLK, _R, _C), 1)
        d2 = jax.lax.broadcasted_iota(jnp.int32, (_BLK, _R, _C), 2)
        mask = (
            (d0 == _DST[0] - dst_blk * _BLK) & (d1 == _DST[1]) & (d2 == _DST[2])
        )
        o_ref[...] = jnp.where(mask, s, 2.0 * jnp.log(x_ref[...]))

    @pl.when(i != dst_blk)
    def _():
        o_ref[...] = 2.0 * jnp.log(x_ref[...])


def kernel(x):
    xr = x.reshape(_LEAD, _R, _C)
    out = pl.pallas_call(
        _ew_kernel,
        grid=(_LEAD // _BLK,),
        in_specs=[
            pl.BlockSpec((_BLK, _R, _C), lambda i: (i, 0, 0)),
            pl.BlockSpec((1, 8, _C), lambda i: (_SRC[0], 0, 0)),
        ],
        out_specs=pl.BlockSpec((_BLK, _R, _C), lambda i: (i, 0, 0)),
        out_shape=jax.ShapeDtypeStruct((_LEAD, _R, _C), x.dtype),
    )(xr, xr)
    return out.reshape(x.shape)
